# baseline (speedup 1.0000x reference)
import functools

import jax
import jax.numpy as jnp
from jax import lax
from jax.experimental import pallas as pl
from jax.experimental.pallas import tpu as pltpu
from jax.experimental.pallas import tpu_sc as plsc

_NUM_MOTIONS = 100
_MAX_T = 2048
_B = 4096
_NC, _NS, _L = 2, 16, 16
_NW = _NC * _NS
_BPW = _B // _NW

_DIMS = (29, 29, 45, 60, 45, 45)


def _sc_gather(wm, ts, totals, tables):
    mesh = plsc.VectorSubcoreMesh(core_axis_name="c", subcore_axis_name="s",
                                  num_cores=_NC, num_subcores=_NS)
    out_type = tuple(
        jax.ShapeDtypeStruct((_B, d), jnp.float32) for d in _DIMS
    )
    scratch = [
        pltpu.VMEM((_BPW,), jnp.int32),
        pltpu.VMEM((_BPW,), jnp.int32),
        pltpu.VMEM((_BPW,), jnp.int32),
        pltpu.VMEM((_BPW,), jnp.int32),
    ] + [pltpu.VMEM((_BPW, d), jnp.float32) for d in _DIMS] + [
        pltpu.SemaphoreType.DMA,
    ]

    @functools.partial(
        pl.kernel, mesh=mesh, out_type=out_type, scratch_types=scratch,
        compiler_params=pltpu.CompilerParams(use_tc_tiling_on_sc=False))
    def k(wm_hbm, ts_hbm, tot_hbm, t0, t1, t2, t3, t4, t5,
          o0, o1, o2, o3, o4, o5,
          wm_v, ts_v, totg_v, idx_v, r0, r1, r2, r3, r4, r5, sem):
        wid = lax.axis_index("s") * _NC + lax.axis_index("c")
        base = wid * _BPW
        pltpu.sync_copy(wm_hbm.at[pl.ds(base, _BPW)], wm_v)
        pltpu.sync_copy(ts_hbm.at[pl.ds(base, _BPW)], ts_v)
        pltpu.async_copy(tot_hbm.at[wm_v], totg_v, sem).wait()
        for j in range(_BPW // _L):
            sl = pl.ds(j * _L, _L)
            wm16 = jnp.minimum(wm_v[sl], _NUM_MOTIONS - 1)
            tot16 = totg_v[sl]
            ts16 = jnp.minimum(ts_v[sl], tot16 - 1)
            idx_v[sl] = wm16 * _MAX_T + ts16
        copies = [
            pltpu.async_copy(tbl.at[idx_v], buf, sem)
            for tbl, buf in ((t0, r0), (t1, r1), (t2, r2),
                             (t3, r3), (t4, r4), (t5, r5))
        ]
        for c in copies:
            c.wait()
        for buf, out in ((r0, o0), (r1, o1), (r2, o2),
                         (r3, o3), (r4, o4), (r5, o5)):
            pltpu.sync_copy(buf, out.at[pl.ds(base, _BPW)])

    return k(wm, ts, totals, *tables)


def _elu(h):
    return jnp.where(h > 0, h, jnp.exp(jnp.minimum(h, 0.0)) - 1.0)


def _mlp_body(x_ref, w1_ref, b1_ref, w2_ref, b2_ref, w3_ref, b3_ref,
              w4_ref, b4_ref, o_ref):
    h = x_ref[...]
    h = _elu(jnp.dot(h, w1_ref[...], preferred_element_type=jnp.float32)
             + b1_ref[...])
    h = _elu(jnp.dot(h, w2_ref[...], preferred_element_type=jnp.float32)
             + b2_ref[...])
    h = _elu(jnp.dot(h, w3_ref[...], preferred_element_type=jnp.float32)
             + b3_ref[...])
    o_ref[...] = (jnp.dot(h, w4_ref[...], preferred_element_type=jnp.float32)
                  + b4_ref[...])


def _mlp(x, W1, b1, W2, b2, W3, b3, W4, b4):
    bm = 512
    obs = x.shape[1]
    act = W4.shape[1]
    grid = (x.shape[0] // bm,)
    b1, b2, b3, b4 = (b.reshape(1, -1) for b in (b1, b2, b3, b4))

    def _full(w):
        return pl.BlockSpec(w.shape, lambda i: (0,) * w.ndim)

    return pl.pallas_call(
        _mlp_body,
        grid=grid,
        in_specs=[pl.BlockSpec((bm, obs), lambda i: (i, 0)),
                  _full(W1), _full(b1), _full(W2), _full(b2),
                  _full(W3), _full(b3), _full(W4), _full(b4)],
        out_specs=pl.BlockSpec((bm, act), lambda i: (i, 0)),
        out_shape=jax.ShapeDtypeStruct((x.shape[0], act), jnp.float32),
    )(x, W1, b1, W2, b2, W3, b3, W4, b4)


def kernel(x, which_motion, time_step, joint_pos, joint_vel, body_pos_w,
           body_quat_w, body_lin_vel_w, body_ang_vel_w, time_step_totals,
           W1, b1, W2, b2, W3, b3, W4, b4):
    wm = which_motion.astype(jnp.int32).reshape(-1)
    ts = time_step.astype(jnp.int32).reshape(-1)
    nm_t = _NUM_MOTIONS * _MAX_T
    tables = (
        joint_pos.reshape(nm_t, 29),
        joint_vel.reshape(nm_t, 29),
        body_pos_w.reshape(nm_t, 45),
        body_quat_w.reshape(nm_t, 60),
        body_lin_vel_w.reshape(nm_t, 45),
        body_ang_vel_w.reshape(nm_t, 45),
    )
    g0, g1, g2, g3, g4, g5 = _sc_gather(
        wm, ts, time_step_totals.astype(jnp.int32), tables)
    action = _mlp(x, W1, b1, W2, b2, W3, b3, W4, b4)
    return (
        action,
        g0,
        g1,
        g2.reshape(_B, 15, 3),
        g3.reshape(_B, 15, 4),
        g4.reshape(_B, 15, 3),
        g5.reshape(_B, 15, 3),
    )


# speedup vs baseline: 5.6837x; 5.6837x over previous
import functools

import jax
import jax.numpy as jnp
from jax import lax
from jax.experimental import pallas as pl
from jax.experimental.pallas import tpu as pltpu
from jax.experimental.pallas import tpu_sc as plsc

_NM = 100
_MT = 2048
_B = 4096
_NC, _NS, _L = 2, 16, 16
_NW = _NC * _NS
_BPW = _B // _NW

_PLANE = _NM * _MT
_NF = 253
_QM, _QB, _QTG, _QC = 15 * 16 * 4 * 128, 16 * 4 * 128, 4 * 128, 128


def _sc_gather(wm, ts, totals, jp_f, jv_f, pos_f, lin_f, ang_f, quat_f):
    mesh = plsc.VectorSubcoreMesh(core_axis_name="c", subcore_axis_name="s",
                                  num_cores=_NC, num_subcores=_NS)
    out_type = jax.ShapeDtypeStruct((_NW * _NF * _BPW,), jnp.float32)
    scratch = [
        pltpu.VMEM((_BPW,), jnp.int32),
        pltpu.VMEM((_BPW,), jnp.int32),
        pltpu.VMEM((_BPW,), jnp.int32),
        pltpu.VMEM((_BPW,), jnp.int32),
        pltpu.VMEM((_BPW,), jnp.int32),
        pltpu.VMEM((29 * _BPW,), jnp.int32),
        pltpu.VMEM((45 * _BPW,), jnp.int32),
        pltpu.VMEM((60 * _BPW,), jnp.int32),
        pltpu.VMEM((_NF * _BPW,), jnp.float32),
        pltpu.SemaphoreType.DMA,
    ]

    @functools.partial(
        pl.kernel, mesh=mesh, out_type=out_type, scratch_types=scratch,
        compiler_params=pltpu.CompilerParams(use_tc_tiling_on_sc=False))
    def k(wm_hbm, ts_hbm, tot_hbm, jp_hbm, jv_hbm, pos_hbm, lin_hbm, ang_hbm,
          q_hbm, out_hbm,
          wm_v, ts_v, totg_v, ridx_v, qb_v, ixj_v, ixp_v, ixq_v, data_v, sem):
        wid = lax.axis_index("s") * _NC + lax.axis_index("c")
        base = wid * _BPW
        pltpu.sync_copy(wm_hbm.at[pl.ds(base, _BPW)], wm_v)
        pltpu.sync_copy(ts_hbm.at[pl.ds(base, _BPW)], ts_v)
        pltpu.async_copy(tot_hbm.at[wm_v], totg_v, sem).wait()
        for j8 in range(_BPW // _L):
            sl = pl.ds(j8 * _L, _L)
            wm16 = jnp.minimum(wm_v[sl], _NM - 1)
            ts16 = jnp.minimum(ts_v[sl], totg_v[sl] - 1)
            ridx_v[sl] = wm16 * _MT + ts16
            qb_v[sl] = (wm16 * _QM + (ts16 >> 7) * _QTG + (ts16 & 127))

        def jbody(j, _):
            for j8 in range(_BPW // _L):
                sl = pl.ds(j8 * _L, _L)
                ixj_v[pl.ds(j * _BPW + j8 * _L, _L)] = ridx_v[sl] + j * _PLANE
            return 0
        lax.fori_loop(0, 29, jbody, 0, unroll=False)

        def pbody(f, _):
            for j8 in range(_BPW // _L):
                sl = pl.ds(j8 * _L, _L)
                ixp_v[pl.ds(f * _BPW + j8 * _L, _L)] = ridx_v[sl] + f * _PLANE
            return 0
        lax.fori_loop(0, 45, pbody, 0, unroll=False)

        def qbody(f, _):
            off = (f >> 2) * _QB + (f & 3) * _QC
            for j8 in range(_BPW // _L):
                sl = pl.ds(j8 * _L, _L)
                ixq_v[pl.ds(f * _BPW + j8 * _L, _L)] = qb_v[sl] + off
            return 0
        lax.fori_loop(0, 60, qbody, 0, unroll=False)

        o_jp, o_jv = 0, 29 * _BPW
        o_pos, o_lin = 58 * _BPW, 103 * _BPW
        o_ang, o_q = 148 * _BPW, 193 * _BPW
        copies = [
            pltpu.async_copy(jp_hbm.at[ixj_v], data_v.at[pl.ds(o_jp, 29 * _BPW)], sem),
            pltpu.async_copy(jv_hbm.at[ixj_v], data_v.at[pl.ds(o_jv, 29 * _BPW)], sem),
            pltpu.async_copy(pos_hbm.at[ixp_v], data_v.at[pl.ds(o_pos, 45 * _BPW)], sem),
            pltpu.async_copy(lin_hbm.at[ixp_v], data_v.at[pl.ds(o_lin, 45 * _BPW)], sem),
            pltpu.async_copy(ang_hbm.at[ixp_v], data_v.at[pl.ds(o_ang, 45 * _BPW)], sem),
            pltpu.async_copy(q_hbm.at[ixq_v], data_v.at[pl.ds(o_q, 60 * _BPW)], sem),
        ]
        for c in copies:
            c.wait()
        pltpu.sync_copy(data_v, out_hbm.at[pl.ds(wid * _NF * _BPW, _NF * _BPW)])

    return k(wm, ts, totals, jp_f, jv_f, pos_f, lin_f, ang_f, quat_f)


def _elu(h):
    return jnp.where(h > 0, h, jnp.exp(jnp.minimum(h, 0.0)) - 1.0)


def _mlp_body(x_ref, w1_ref, b1_ref, w2_ref, b2_ref, w3_ref, b3_ref,
              w4_ref, b4_ref, o_ref):
    h = x_ref[...]
    h = _elu(jnp.dot(h, w1_ref[...], preferred_element_type=jnp.float32)
             + b1_ref[...])
    h = _elu(jnp.dot(h, w2_ref[...], preferred_element_type=jnp.float32)
             + b2_ref[...])
    h = _elu(jnp.dot(h, w3_ref[...], preferred_element_type=jnp.float32)
             + b3_ref[...])
    o_ref[...] = (jnp.dot(h, w4_ref[...], preferred_element_type=jnp.float32)
                  + b4_ref[...])


def _mlp(x, W1, b1, W2, b2, W3, b3, W4, b4):
    bm = 512
    obs = x.shape[1]
    act = W4.shape[1]
    grid = (x.shape[0] // bm,)
    b1, b2, b3, b4 = (b.reshape(1, -1) for b in (b1, b2, b3, b4))

    def _full(w):
        return pl.BlockSpec(w.shape, lambda i: (0,) * w.ndim)

    return pl.pallas_call(
        _mlp_body,
        grid=grid,
        in_specs=[pl.BlockSpec((bm, obs), lambda i: (i, 0)),
                  _full(W1), _full(b1), _full(W2), _full(b2),
                  _full(W3), _full(b3), _full(W4), _full(b4)],
        out_specs=pl.BlockSpec((bm, act), lambda i: (i, 0)),
        out_shape=jax.ShapeDtypeStruct((x.shape[0], act), jnp.float32),
    )(x, W1, b1, W2, b2, W3, b3, W4, b4)


def kernel(x, which_motion, time_step, joint_pos, joint_vel, body_pos_w,
           body_quat_w, body_lin_vel_w, body_ang_vel_w, time_step_totals,
           W1, b1, W2, b2, W3, b3, W4, b4):
    wm = which_motion.astype(jnp.int32).reshape(-1)
    ts = time_step.astype(jnp.int32).reshape(-1)
    jp_f = jnp.transpose(joint_pos, (2, 0, 1)).reshape(-1)
    jv_f = jnp.transpose(joint_vel, (2, 0, 1)).reshape(-1)
    pos_f = jnp.transpose(body_pos_w, (2, 3, 0, 1)).reshape(-1)
    lin_f = jnp.transpose(body_lin_vel_w, (2, 3, 0, 1)).reshape(-1)
    ang_f = jnp.transpose(body_ang_vel_w, (2, 3, 0, 1)).reshape(-1)
    quat_f = (body_quat_w.reshape(_NM, 16, 128, 15, 4)
              .transpose(0, 3, 1, 4, 2).reshape(-1))

    flat = _sc_gather(wm, ts, time_step_totals.astype(jnp.int32),
                      jp_f, jv_f, pos_f, lin_f, ang_f, quat_f)
    arr = flat.reshape(_NW, _NF, _BPW)

    def take(lo, n):
        return arr[:, lo:lo + n, :].transpose(0, 2, 1).reshape(_B, n)

    action = _mlp(x, W1, b1, W2, b2, W3, b3, W4, b4)
    return (
        action,
        take(0, 29),
        take(29, 29),
        take(58, 45).reshape(_B, 15, 3),
        take(193, 60).reshape(_B, 15, 4),
        take(103, 45).reshape(_B, 15, 3),
        take(148, 45).reshape(_B, 15, 3),
    )
